# Initial kernel scaffold; baseline (speedup 1.0000x reference)
import jax
import jax.numpy as jnp
from jax.experimental import pallas as pl


def kernel(x, edge_index, W1_l, b1_l, W1_r, W2_l, b2_l, W2_r):
    raise NotImplementedError("write your pallas kernel here")


# speedup vs baseline: 5.6365x; 5.6365x over previous
import functools

import jax
import jax.numpy as jnp
from jax import lax
from jax.experimental import pallas as pl
from jax.experimental.pallas import tpu as pltpu
from jax.experimental.pallas import tpu_sc as plsc

N = 10000
E = 320000
D = 128
HD = D // 2
NC = 2
NS = 16
EPW = E // NS
CH = 80
NCH = EPW // CH
RPS = 624
TAIL = N - RPS * NS

_MESH = plsc.VectorSubcoreMesh(
    core_axis_name="c", subcore_axis_name="s", num_cores=NC, num_subcores=NS
)


def _sc_body(with_deg, *refs):
    if with_deg:
        (table0, table1, src3, dst3, out0, out1, dego,
         src_v, dst_v, rows_v, ones_v, zrow_v, zdeg_v,
         acc_sh, deg_sh, gsem) = refs
    else:
        (table0, table1, src3, dst3, out0, out1,
         src_v, dst_v, rows_v, zrow_v,
         acc_sh, gsem) = refs

    c = lax.axis_index("c")
    s = lax.axis_index("s")

    zeros16 = jnp.zeros((16,), jnp.float32)
    start = pl.multiple_of(s * RPS, 16)

    def _zrow(i, _):
        for k in range(HD // 16):
            zrow_v[i, pl.ds(k * 16, 16)] = zeros16
        return 0

    lax.fori_loop(0, 104, _zrow, 0)

    def _zacc(i, _):
        pltpu.sync_copy(zrow_v, acc_sh.at[pl.ds(pl.multiple_of(start + i * 104, 8), 104)])
        return 0

    lax.fori_loop(0, RPS // 104, _zacc, 0)

    @pl.when(s == NS - 1)
    def _():
        pltpu.sync_copy(zrow_v.at[pl.ds(0, TAIL)], acc_sh.at[pl.ds(RPS * NS, TAIL)])

    if with_deg:
        def _zdeg(i, _):
            zdeg_v[i] = zeros16
            return 0

        lax.fori_loop(0, 104, _zdeg, 0)

        ones16 = jnp.ones((16,), jnp.float32)

        def _ones(i, _):
            ones_v[i] = ones16
            return 0

        lax.fori_loop(0, CH, _ones, 0)

        @pl.when(c == 0)
        def _():
            def _zdacc(i, _):
                pltpu.sync_copy(
                    zdeg_v, deg_sh.at[pl.ds(pl.multiple_of(start + i * 104, 8), 104)])
                return 0

            lax.fori_loop(0, RPS // 104, _zdacc, 0)

            @pl.when(s == NS - 1)
            def _():
                pltpu.sync_copy(zdeg_v.at[pl.ds(0, TAIL)],
                                deg_sh.at[pl.ds(RPS * NS, TAIL)])

    pltpu.sync_copy(src3.at[s], src_v)
    pltpu.sync_copy(dst3.at[s], dst_v)

    plsc.subcore_barrier()

    def _run(table, count_deg):
        def _chunk(ci, _):
            pltpu.async_copy(table.at[src_v.at[ci]], rows_v, gsem).wait()
            pltpu.sync_copy(rows_v, acc_sh.at[dst_v.at[ci]], add=True)
            if count_deg:
                pltpu.sync_copy(ones_v, deg_sh.at[dst_v.at[ci]], add=True)
            return 0

        lax.fori_loop(0, NCH, _chunk, 0)

    @pl.when(c == 0)
    def _():
        _run(table0, with_deg)

    @pl.when(c == 1)
    def _():
        _run(table1, False)

    plsc.subcore_barrier()

    def _share_copy(src_sh, dst_hbm):
        pltpu.sync_copy(src_sh.at[pl.ds(start, RPS)], dst_hbm.at[pl.ds(start, RPS)])

        @pl.when(s == NS - 1)
        def _():
            pltpu.sync_copy(src_sh.at[pl.ds(RPS * NS, TAIL)],
                            dst_hbm.at[pl.ds(RPS * NS, TAIL)])

    @pl.when(c == 0)
    def _():
        _share_copy(acc_sh, out0)
        if with_deg:
            _share_copy(deg_sh, dego)

    @pl.when(c == 1)
    def _():
        _share_copy(acc_sh, out1)


def _make_sc(with_deg):
    f32 = jnp.float32
    outs = [jax.ShapeDtypeStruct((N, HD), f32), jax.ShapeDtypeStruct((N, HD), f32)]
    scratch = [
        pltpu.VMEM((NCH, CH), jnp.int32),
        pltpu.VMEM((NCH, CH), jnp.int32),
        pltpu.VMEM((CH, HD), f32),
    ]
    if with_deg:
        outs += [jax.ShapeDtypeStruct((N, 16), f32)]
        scratch += [pltpu.VMEM((CH, 16), f32)]
    scratch += [pltpu.VMEM((104, HD), f32)]
    if with_deg:
        scratch += [pltpu.VMEM((104, 16), f32)]
    scratch += [pltpu.VMEM_SHARED((N, HD), f32)]
    if with_deg:
        scratch += [pltpu.VMEM_SHARED((N, 16), f32)]
    scratch += [pltpu.SemaphoreType.DMA]

    return pl.kernel(
        functools.partial(_sc_body, with_deg),
        out_type=tuple(outs),
        mesh=_MESH,
        scratch_types=scratch,
        compiler_params=pltpu.CompilerParams(use_tc_tiling_on_sc=False),
    )


_SC_L1 = _make_sc(True)
_SC_L2 = _make_sc(False)

_BLK = 1000


def _tc_body1(x_ref, p0_ref, p1_ref, dg_ref, wl_ref, b_ref, wr_ref,
              o0_ref, o1_ref):
    agg = jnp.concatenate([p0_ref[...], p1_ref[...]], axis=1)
    mean = agg / jnp.maximum(dg_ref[:, 0:1], 1.0)
    dn = (((1,), (1,)), ((), ()))
    h = lax.dot_general(mean, wl_ref[...], dn, preferred_element_type=jnp.float32)
    h = h + b_ref[...] + lax.dot_general(
        x_ref[...], wr_ref[...], dn, preferred_element_type=jnp.float32)
    a = jax.nn.relu(h)
    o0_ref[...] = a[:, :HD]
    o1_ref[...] = a[:, HD:]


def _tc_body2(h0_ref, h1_ref, q0_ref, q1_ref, dg_ref, wl_ref, b_ref, wr_ref,
              o_ref):
    xs = jnp.concatenate([h0_ref[...], h1_ref[...]], axis=1)
    agg = jnp.concatenate([q0_ref[...], q1_ref[...]], axis=1)
    mean = agg / jnp.maximum(dg_ref[:, 0:1], 1.0)
    dn = (((1,), (1,)), ((), ()))
    h = lax.dot_general(mean, wl_ref[...], dn, preferred_element_type=jnp.float32)
    h = h + b_ref[...] + lax.dot_general(
        xs, wr_ref[...], dn, preferred_element_type=jnp.float32)
    o_ref[...] = jax.nn.sigmoid(h)


_row = pl.BlockSpec((_BLK, D), lambda i: (i, 0))
_half = pl.BlockSpec((_BLK, HD), lambda i: (i, 0))
_dgs = pl.BlockSpec((_BLK, 16), lambda i: (i, 0))
_full = pl.BlockSpec((D, D), lambda i: (0, 0))
_bias = pl.BlockSpec((1, D), lambda i: (0, 0))

_TC_L1 = pl.pallas_call(
    _tc_body1,
    grid=(N // _BLK,),
    in_specs=[_row, _half, _half, _dgs, _full, _bias, _full],
    out_specs=[_half, _half],
    out_shape=[jax.ShapeDtypeStruct((N, HD), jnp.float32),
               jax.ShapeDtypeStruct((N, HD), jnp.float32)],
)

_TC_L2 = pl.pallas_call(
    _tc_body2,
    grid=(N // _BLK,),
    in_specs=[_half, _half, _half, _half, _dgs, _full, _bias, _full],
    out_specs=_row,
    out_shape=jax.ShapeDtypeStruct((N, D), jnp.float32),
)


def kernel(x, edge_index, W1_l, b1_l, W1_r, W2_l, b2_l, W2_r):
    src3 = edge_index[0].astype(jnp.int32).reshape(NS, NCH, CH)
    dst3 = edge_index[1].astype(jnp.int32).reshape(NS, NCH, CH)
    x0 = x[:, :HD]
    x1 = x[:, HD:]

    p0, p1, dg = _SC_L1(x0, x1, src3, dst3)
    h0, h1 = _TC_L1(x, p0, p1, dg, W1_l, b1_l.reshape(1, D), W1_r)
    q0, q1 = _SC_L2(h0, h1, src3, dst3)
    return _TC_L2(h0, h1, q0, q1, dg, W2_l, b2_l.reshape(1, D), W2_r)


# speedup vs baseline: 9.7881x; 1.7366x over previous
import functools

import jax
import jax.numpy as jnp
from jax import lax
from jax.experimental import pallas as pl
from jax.experimental.pallas import tpu as pltpu
from jax.experimental.pallas import tpu_sc as plsc

N = 10000
E = 320000
D = 128
HD = D // 2
NC = 2
NS = 16
EPW = E // NS
CH = 80
NCH = EPW // CH
RPS = 624
TAIL = N - RPS * NS
K = 5

_MESH = plsc.VectorSubcoreMesh(
    core_axis_name="c", subcore_axis_name="s", num_cores=NC, num_subcores=NS
)


def _sc_body(with_deg, *refs):
    if with_deg:
        (table0, table1, src3, dst3, out0, out1, dego,
         src_v, dst_v, rows_v, ones_v, zrow_v, zdeg_v,
         acc_sh, deg_sh, *sems) = refs
    else:
        (table0, table1, src3, dst3, out0, out1,
         src_v, dst_v, rows_v, zrow_v,
         acc_sh, *sems) = refs
    gsems = sems[:K]
    ssems = sems[K:2 * K]
    dsems = sems[2 * K:]

    c = lax.axis_index("c")
    s = lax.axis_index("s")

    zeros16 = jnp.zeros((16,), jnp.float32)
    start = pl.multiple_of(s * RPS, 16)

    def _zrow(i, _):
        for k in range(HD // 16):
            zrow_v[i, pl.ds(k * 16, 16)] = zeros16
        return 0

    lax.fori_loop(0, 104, _zrow, 0)

    def _zacc(i, _):
        pltpu.sync_copy(zrow_v, acc_sh.at[pl.ds(pl.multiple_of(start + i * 104, 8), 104)])
        return 0

    lax.fori_loop(0, RPS // 104, _zacc, 0)

    @pl.when(s == NS - 1)
    def _():
        pltpu.sync_copy(zrow_v.at[pl.ds(0, TAIL)], acc_sh.at[pl.ds(RPS * NS, TAIL)])

    if with_deg:
        def _zdeg(i, _):
            zdeg_v[i] = zeros16
            return 0

        lax.fori_loop(0, 104, _zdeg, 0)

        ones16 = jnp.ones((16,), jnp.float32)

        def _ones(i, _):
            ones_v[i] = ones16
            return 0

        lax.fori_loop(0, CH, _ones, 0)

        @pl.when(c == 0)
        def _():
            def _zdacc(i, _):
                pltpu.sync_copy(
                    zdeg_v, deg_sh.at[pl.ds(pl.multiple_of(start + i * 104, 8), 104)])
                return 0

            lax.fori_loop(0, RPS // 104, _zdacc, 0)

            @pl.when(s == NS - 1)
            def _():
                pltpu.sync_copy(zdeg_v.at[pl.ds(0, TAIL)],
                                deg_sh.at[pl.ds(RPS * NS, TAIL)])

    pltpu.sync_copy(src3.at[s], src_v)
    pltpu.sync_copy(dst3.at[s], dst_v)

    plsc.subcore_barrier()

    def _run(table, count_deg):
        def _iter(it, _):
            base = it * K
            gd = [
                pltpu.async_copy(table.at[src_v.at[base + k]],
                                 rows_v.at[k], gsems[k])
                for k in range(K)
            ]
            sd = []
            dd = []
            for k in range(K):
                gd[k].wait()
                sd.append(pltpu.async_copy(
                    rows_v.at[k], acc_sh.at[dst_v.at[base + k]], ssems[k],
                    add=True))
                if count_deg:
                    dd.append(pltpu.async_copy(
                        ones_v, deg_sh.at[dst_v.at[base + k]], dsems[k],
                        add=True))
            for d in sd + dd:
                d.wait()
            return 0

        lax.fori_loop(0, NCH // K, _iter, 0)

    @pl.when(c == 0)
    def _():
        _run(table0, with_deg)

    @pl.when(c == 1)
    def _():
        _run(table1, False)

    plsc.subcore_barrier()

    def _share_copy(src_sh, dst_hbm):
        pltpu.sync_copy(src_sh.at[pl.ds(start, RPS)], dst_hbm.at[pl.ds(start, RPS)])

        @pl.when(s == NS - 1)
        def _():
            pltpu.sync_copy(src_sh.at[pl.ds(RPS * NS, TAIL)],
                            dst_hbm.at[pl.ds(RPS * NS, TAIL)])

    @pl.when(c == 0)
    def _():
        _share_copy(acc_sh, out0)
        if with_deg:
            _share_copy(deg_sh, dego)

    @pl.when(c == 1)
    def _():
        _share_copy(acc_sh, out1)


def _make_sc(with_deg):
    f32 = jnp.float32
    outs = [jax.ShapeDtypeStruct((N, HD), f32), jax.ShapeDtypeStruct((N, HD), f32)]
    scratch = [
        pltpu.VMEM((NCH, CH), jnp.int32),
        pltpu.VMEM((NCH, CH), jnp.int32),
        pltpu.VMEM((K, CH, HD), f32),
    ]
    if with_deg:
        outs += [jax.ShapeDtypeStruct((N, 16), f32)]
        scratch += [pltpu.VMEM((CH, 16), f32)]
    scratch += [pltpu.VMEM((104, HD), f32)]
    if with_deg:
        scratch += [pltpu.VMEM((104, 16), f32)]
    scratch += [pltpu.VMEM_SHARED((N, HD), f32)]
    if with_deg:
        scratch += [pltpu.VMEM_SHARED((N, 16), f32)]
    nsem = 3 * K if with_deg else 2 * K
    scratch += [pltpu.SemaphoreType.DMA] * nsem

    return pl.kernel(
        functools.partial(_sc_body, with_deg),
        out_type=tuple(outs),
        mesh=_MESH,
        scratch_types=scratch,
        compiler_params=pltpu.CompilerParams(use_tc_tiling_on_sc=False),
    )


_SC_L1 = _make_sc(True)
_SC_L2 = _make_sc(False)

_BLK = 1000


def _tc_body1(x_ref, p0_ref, p1_ref, dg_ref, wl_ref, b_ref, wr_ref,
              o0_ref, o1_ref):
    agg = jnp.concatenate([p0_ref[...], p1_ref[...]], axis=1)
    mean = agg / jnp.maximum(dg_ref[:, 0:1], 1.0)
    dn = (((1,), (1,)), ((), ()))
    h = lax.dot_general(mean, wl_ref[...], dn, preferred_element_type=jnp.float32)
    h = h + b_ref[...] + lax.dot_general(
        x_ref[...], wr_ref[...], dn, preferred_element_type=jnp.float32)
    a = jax.nn.relu(h)
    o0_ref[...] = a[:, :HD]
    o1_ref[...] = a[:, HD:]


def _tc_body2(h0_ref, h1_ref, q0_ref, q1_ref, dg_ref, wl_ref, b_ref, wr_ref,
              o_ref):
    xs = jnp.concatenate([h0_ref[...], h1_ref[...]], axis=1)
    agg = jnp.concatenate([q0_ref[...], q1_ref[...]], axis=1)
    mean = agg / jnp.maximum(dg_ref[:, 0:1], 1.0)
    dn = (((1,), (1,)), ((), ()))
    h = lax.dot_general(mean, wl_ref[...], dn, preferred_element_type=jnp.float32)
    h = h + b_ref[...] + lax.dot_general(
        xs, wr_ref[...], dn, preferred_element_type=jnp.float32)
    o_ref[...] = jax.nn.sigmoid(h)


_row = pl.BlockSpec((_BLK, D), lambda i: (i, 0))
_half = pl.BlockSpec((_BLK, HD), lambda i: (i, 0))
_dgs = pl.BlockSpec((_BLK, 16), lambda i: (i, 0))
_full = pl.BlockSpec((D, D), lambda i: (0, 0))
_bias = pl.BlockSpec((1, D), lambda i: (0, 0))

_TC_L1 = pl.pallas_call(
    _tc_body1,
    grid=(N // _BLK,),
    in_specs=[_row, _half, _half, _dgs, _full, _bias, _full],
    out_specs=[_half, _half],
    out_shape=[jax.ShapeDtypeStruct((N, HD), jnp.float32),
               jax.ShapeDtypeStruct((N, HD), jnp.float32)],
)

_TC_L2 = pl.pallas_call(
    _tc_body2,
    grid=(N // _BLK,),
    in_specs=[_half, _half, _half, _half, _dgs, _full, _bias, _full],
    out_specs=_row,
    out_shape=jax.ShapeDtypeStruct((N, D), jnp.float32),
)


def kernel(x, edge_index, W1_l, b1_l, W1_r, W2_l, b2_l, W2_r):
    src3 = edge_index[0].astype(jnp.int32).reshape(NS, NCH, CH)
    dst3 = edge_index[1].astype(jnp.int32).reshape(NS, NCH, CH)
    x0 = x[:, :HD]
    x1 = x[:, HD:]

    p0, p1, dg = _SC_L1(x0, x1, src3, dst3)
    h0, h1 = _TC_L1(x, p0, p1, dg, W1_l, b1_l.reshape(1, D), W1_r)
    q0, q1 = _SC_L2(h0, h1, src3, dst3)
    return _TC_L2(h0, h1, q0, q1, dg, W2_l, b2_l.reshape(1, D), W2_r)
